# Initial kernel scaffold; baseline (speedup 1.0000x reference)
import jax
import jax.numpy as jnp
from jax.experimental import pallas as pl


def kernel(numeric_features, cat_features, tables, W1, b1, g1, be1, W2, b2, g2, be2, W3, b3, g3, be3, W4, b4):
    raise NotImplementedError("write your pallas kernel here")


# speedup vs baseline: 6.8999x; 6.8999x over previous
import functools

import jax
import jax.numpy as jnp
from jax import lax
from jax.experimental import pallas as pl
from jax.experimental.pallas import tpu as pltpu
from jax.experimental.pallas import tpu_sc as plsc

EPS = 1e-5
GW = 128
TB = 512


def _sc_gather(tables_flat, idx2d):
    n_idx = idx2d.shape[1]
    d = tables_flat.shape[1]
    mesh = plsc.VectorSubcoreMesh(core_axis_name="core", subcore_axis_name="subcore")

    @functools.partial(
        pl.kernel,
        out_type=jax.ShapeDtypeStruct((n_idx, d), tables_flat.dtype),
        mesh=mesh,
        compiler_params=pltpu.CompilerParams(use_tc_tiling_on_sc=False),
    )
    def gather_kernel(tab_hbm, idx_hbm, out_hbm):
        def body(i_vmem, o_vmem):
            pltpu.sync_copy(tab_hbm.at[i_vmem.at[0]], o_vmem)

        pltpu.emit_pipeline(
            body,
            grid=(n_idx // GW,),
            in_specs=[pl.BlockSpec((1, GW), index_map=lambda i: (0, i))],
            out_specs=[pl.BlockSpec((GW, d), index_map=lambda i: (i, 0))],
            core_axis_name=("core", "subcore"),
            dimension_semantics=(pltpu.PARALLEL,),
        )(idx_hbm, out_hbm)

    return gather_kernel(tables_flat, idx2d)


def _stats_update(s_ref, h):
    st = jnp.concatenate(
        [jnp.sum(h, axis=0, keepdims=True), jnp.sum(h * h, axis=0, keepdims=True)], axis=0
    )

    @pl.when(pl.program_id(0) == 0)
    def _():
        s_ref[...] = st

    @pl.when(pl.program_id(0) != 0)
    def _():
        s_ref[...] += st


def _layer1_body(num_ref, emb_ref, wn_ref, we_ref, b_ref, h_ref, s_ref):
    h = jnp.dot(num_ref[...], wn_ref[...], preferred_element_type=jnp.float32)
    h = h + jnp.dot(emb_ref[...], we_ref[...], preferred_element_type=jnp.float32)
    h = h + b_ref[...]
    h_ref[...] = h
    _stats_update(s_ref, h)


def _bn_relu(h, s, g, be, batch):
    mu = s[0:1, :] * (1.0 / batch)
    var = s[1:2, :] * (1.0 / batch) - mu * mu
    scale = g * lax.rsqrt(var + EPS)
    shift = be - mu * scale
    return jnp.maximum(h * scale + shift, 0.0)


def _mid_body(h_ref, s_ref, g_ref, be_ref, w_ref, b_ref, o_ref, so_ref, *, batch):
    a = _bn_relu(h_ref[...], s_ref[...], g_ref[...], be_ref[...], batch)
    h = jnp.dot(a, w_ref[...], preferred_element_type=jnp.float32) + b_ref[...]
    o_ref[...] = h
    _stats_update(so_ref, h)


def _last_body(h_ref, s_ref, g_ref, be_ref, w_ref, b_ref, o_ref, *, batch):
    a = _bn_relu(h_ref[...], s_ref[...], g_ref[...], be_ref[...], batch)
    o_ref[...] = jnp.dot(a, w_ref[...], preferred_element_type=jnp.float32) + b_ref[...]


def _full(shape):
    return pl.BlockSpec(shape, lambda i: (0,) * len(shape))


def _layer1(num, emb, wn, we, b):
    batch, h1 = num.shape[0], wn.shape[1]
    return pl.pallas_call(
        _layer1_body,
        grid=(batch // TB,),
        in_specs=[
            pl.BlockSpec((TB, num.shape[1]), lambda i: (i, 0)),
            pl.BlockSpec((TB, emb.shape[1]), lambda i: (i, 0)),
            _full(wn.shape),
            _full(we.shape),
            _full(b.shape),
        ],
        out_specs=[
            pl.BlockSpec((TB, h1), lambda i: (i, 0)),
            _full((2, h1)),
        ],
        out_shape=[
            jax.ShapeDtypeStruct((batch, h1), jnp.float32),
            jax.ShapeDtypeStruct((2, h1), jnp.float32),
        ],
    )(num, emb, wn, we, b)


def _mid(h, s, g, be, w, b):
    batch, hout = h.shape[0], w.shape[1]
    return pl.pallas_call(
        functools.partial(_mid_body, batch=batch),
        grid=(batch // TB,),
        in_specs=[
            pl.BlockSpec((TB, h.shape[1]), lambda i: (i, 0)),
            _full(s.shape),
            _full(g.shape),
            _full(be.shape),
            _full(w.shape),
            _full(b.shape),
        ],
        out_specs=[
            pl.BlockSpec((TB, hout), lambda i: (i, 0)),
            _full((2, hout)),
        ],
        out_shape=[
            jax.ShapeDtypeStruct((batch, hout), jnp.float32),
            jax.ShapeDtypeStruct((2, hout), jnp.float32),
        ],
    )(h, s, g, be, w, b)


def _last(h, s, g, be, w, b):
    batch, hout = h.shape[0], w.shape[1]
    return pl.pallas_call(
        functools.partial(_last_body, batch=batch),
        grid=(batch // TB,),
        in_specs=[
            pl.BlockSpec((TB, h.shape[1]), lambda i: (i, 0)),
            _full(s.shape),
            _full(g.shape),
            _full(be.shape),
            _full(w.shape),
            _full(b.shape),
        ],
        out_specs=pl.BlockSpec((TB, hout), lambda i: (i, 0)),
        out_shape=jax.ShapeDtypeStruct((batch, hout), jnp.float32),
    )(h, s, g, be, w, b)


def kernel(numeric_features, cat_features, tables,
           W1, b1, g1, be1, W2, b2, g2, be2, W3, b3, g3, be3, W4, b4):
    batch, num_dim = numeric_features.shape
    f, v, d = tables.shape

    idx = cat_features + (jnp.arange(f, dtype=jnp.int32) * v)[None, :]
    idx2d = idx.reshape(1, batch * f)
    emb = _sc_gather(tables.reshape(f * v, d), idx2d)
    emb = emb.reshape(batch, f * d)

    row = lambda x: x.reshape(1, -1)
    h1, s1 = _layer1(numeric_features, emb, W1[:num_dim], W1[num_dim:], row(b1))
    h2, s2 = _mid(h1, s1, row(g1), row(be1), W2, row(b2))
    h3, s3 = _mid(h2, s2, row(g2), row(be2), W3, row(b3))
    return _last(h3, s3, row(g3), row(be3), W4, row(b4))
